# Initial kernel scaffold; baseline (speedup 1.0000x reference)
import jax
import jax.numpy as jnp
from jax.experimental import pallas as pl


def kernel(x):
    raise NotImplementedError("write your pallas kernel here")


# speedup vs baseline: 10.4650x; 10.4650x over previous
import functools

import jax
import jax.numpy as jnp
from jax.experimental import pallas as pl
from jax.experimental.pallas import tpu as pltpu


def _pool_body(x_ref, o_ref):
    _, _, d, h, w = x_ref.shape
    m = None
    for t in range(2):
        for dd in range(2):
            for hh in range(2):
                v = x_ref[
                    pl.ds(0, 1), pl.ds(t, 1),
                    pl.ds(dd, d // 2, 2), pl.ds(hh, h // 2, 2), :,
                ].reshape(d // 2, h // 2, w)
                m = v if m is None else jnp.maximum(m, v)
    p = jnp.maximum(m, pltpu.roll(m, w - 1, axis=2))
    idx = 2 * jax.lax.broadcasted_iota(jnp.int32, (d // 2, h // 2, w // 2), 2)
    o_ref[0, 0] = jnp.take_along_axis(p, idx, axis=2)


def kernel(x):
    b, c, t, d, h, w = x.shape
    xr = x.reshape(b * c, t, d, h, w)
    out = pl.pallas_call(
        _pool_body,
        grid=(b * c, t // 2),
        in_specs=[
            pl.BlockSpec((1, 2, d, h, w), lambda i, j: (i, j, 0, 0, 0)),
        ],
        out_specs=pl.BlockSpec(
            (1, 1, d // 2, h // 2, w // 2), lambda i, j: (i, j, 0, 0, 0)
        ),
        out_shape=jax.ShapeDtypeStruct(
            (b * c, t // 2, d // 2, h // 2, w // 2), x.dtype
        ),
        compiler_params=pltpu.CompilerParams(
            dimension_semantics=("parallel", "arbitrary"),
        ),
    )(xr)
    return out.reshape(b, c, t // 2, d // 2, h // 2, w // 2)


# speedup vs baseline: 20.2446x; 1.9345x over previous
import functools

import jax
import jax.numpy as jnp
from jax.experimental import pallas as pl
from jax.experimental.pallas import tpu as pltpu


def _pool_body(x_ref, o_ref):
    _, t2, d, h, w = x_ref.shape
    for tp in range(t2 // 2):
        m = None
        for t in range(2):
            for dd in range(2):
                for hh in range(2):
                    v = x_ref[
                        pl.ds(0, 1), pl.ds(2 * tp + t, 1),
                        pl.ds(dd, d // 2, 2), pl.ds(hh, h // 2, 2), :,
                    ].reshape(d // 2, h // 2, w)
                    m = v if m is None else jnp.maximum(m, v)
        p = jnp.maximum(m, pltpu.roll(m, w - 1, axis=2))
        idx = 2 * jax.lax.broadcasted_iota(
            jnp.int32, (d // 2, h // 2, w // 2), 2
        )
        o_ref[0, tp] = jnp.take_along_axis(p, idx, axis=2)


def kernel(x):
    b, c, t, d, h, w = x.shape
    xr = x.reshape(b * c, t, d, h, w)
    tp = 4
    out = pl.pallas_call(
        _pool_body,
        grid=(b * c, t // (2 * tp)),
        in_specs=[
            pl.BlockSpec((1, 2 * tp, d, h, w), lambda i, j: (i, j, 0, 0, 0)),
        ],
        out_specs=pl.BlockSpec(
            (1, tp, d // 2, h // 2, w // 2), lambda i, j: (i, j, 0, 0, 0)
        ),
        out_shape=jax.ShapeDtypeStruct(
            (b * c, t // 2, d // 2, h // 2, w // 2), x.dtype
        ),
        compiler_params=pltpu.CompilerParams(
            dimension_semantics=("parallel", "arbitrary"),
        ),
    )(xr)
    return out.reshape(b, c, t // 2, d // 2, h // 2, w // 2)


# speedup vs baseline: 23.5364x; 1.1626x over previous
import functools

import jax
import jax.numpy as jnp
from jax.experimental import pallas as pl
from jax.experimental.pallas import tpu as pltpu


def _pool_body(x_ref, o_ref):
    _, t2, d, h, w = x_ref.shape
    for tp in range(t2 // 2):
        m = None
        for t in range(2):
            for dd in range(2):
                for hh in range(2):
                    v = x_ref[
                        pl.ds(0, 1), pl.ds(2 * tp + t, 1),
                        pl.ds(dd, d // 2, 2), pl.ds(hh, h // 2, 2), :,
                    ].reshape(d // 2, h // 2, w)
                    m = v if m is None else jnp.maximum(m, v)
        p = jnp.maximum(m, pltpu.roll(m, w - 1, axis=2))
        idx = 2 * jax.lax.broadcasted_iota(
            jnp.int32, (d // 2, h // 2, w // 2), 2
        )
        o_ref[0, tp] = jnp.take_along_axis(p, idx, axis=2)


def kernel(x):
    b, c, t, d, h, w = x.shape
    xr = x.reshape(b * c, t, d, h, w)
    tp = 8
    out = pl.pallas_call(
        _pool_body,
        grid=(b * c, t // (2 * tp)),
        in_specs=[
            pl.BlockSpec((1, 2 * tp, d, h, w), lambda i, j: (i, j, 0, 0, 0)),
        ],
        out_specs=pl.BlockSpec(
            (1, tp, d // 2, h // 2, w // 2), lambda i, j: (i, j, 0, 0, 0)
        ),
        out_shape=jax.ShapeDtypeStruct(
            (b * c, t // 2, d // 2, h // 2, w // 2), x.dtype
        ),
        compiler_params=pltpu.CompilerParams(
            dimension_semantics=("parallel", "arbitrary"),
        ),
    )(xr)
    return out.reshape(b, c, t // 2, d // 2, h // 2, w // 2)


# speedup vs baseline: 24.8598x; 1.0562x over previous
import functools

import jax
import jax.numpy as jnp
from jax.experimental import pallas as pl
from jax.experimental.pallas import tpu as pltpu


def _pool_body(x_ref, o_ref):
    _, t2, d, h, w = x_ref.shape
    for tp in range(t2 // 2):
        m = None
        for t in range(2):
            for dd in range(2):
                for hh in range(2):
                    v = x_ref[
                        pl.ds(0, 1), pl.ds(2 * tp + t, 1),
                        pl.ds(dd, d // 2, 2), pl.ds(hh, h // 2, 2), :,
                    ].reshape(d // 2, h // 2, w)
                    m = v if m is None else jnp.maximum(m, v)
        p = jnp.maximum(m, pltpu.roll(m, w - 1, axis=2))
        idx = 2 * jax.lax.broadcasted_iota(
            jnp.int32, (d // 2, h // 2, w // 2), 2
        )
        o_ref[0, tp] = jnp.take_along_axis(p, idx, axis=2)


def kernel(x):
    b, c, t, d, h, w = x.shape
    xr = x.reshape(b * c, t, d, h, w)
    tp = 16
    out = pl.pallas_call(
        _pool_body,
        grid=(b * c, t // (2 * tp)),
        in_specs=[
            pl.BlockSpec((1, 2 * tp, d, h, w), lambda i, j: (i, j, 0, 0, 0)),
        ],
        out_specs=pl.BlockSpec(
            (1, tp, d // 2, h // 2, w // 2), lambda i, j: (i, j, 0, 0, 0)
        ),
        out_shape=jax.ShapeDtypeStruct(
            (b * c, t // 2, d // 2, h // 2, w // 2), x.dtype
        ),
        compiler_params=pltpu.CompilerParams(
            dimension_semantics=("parallel", "arbitrary"),
        ),
    )(xr)
    return out.reshape(b, c, t // 2, d // 2, h // 2, w // 2)
